# Initial kernel scaffold; baseline (speedup 1.0000x reference)
import jax
import jax.numpy as jnp
from jax.experimental import pallas as pl


def kernel(img, W):
    raise NotImplementedError("write your pallas kernel here")


# speedup vs baseline: 23.2473x; 23.2473x over previous
import jax
import jax.numpy as jnp
from jax.experimental import pallas as pl
from jax.experimental.pallas import tpu as pltpu

PATCH = 16
KTOP = 100
N = 1024
HD = 32
BIG = HD * HD + 2


def _lost_body(feats_ref, w_ref, out_ref, sim_ref, tmp_ref):
    f32 = jnp.float32
    feats = feats_ref[...]
    w = w_ref[...]
    out = jax.lax.dot_general(
        feats, w, (((1,), (0,)), ((), ())),
        precision=jax.lax.Precision.HIGHEST, preferred_element_type=f32)
    sim = jax.lax.dot_general(
        out, out, (((1,), (1,)), ((), ())),
        precision=jax.lax.Precision.HIGHEST, preferred_element_type=f32)
    sim_ref[...] = sim

    iota_r = jax.lax.broadcasted_iota(jnp.int32, (N, N), 0)
    iota_c = jax.lax.broadcasted_iota(jnp.int32, (N, N), 1)
    lane = jax.lax.broadcasted_iota(jnp.int32, (1, N), 1)

    degrees = jnp.sum((sim >= 0).astype(jnp.int32), axis=0, keepdims=True)
    dmin = jnp.min(degrees)
    seed = jnp.min(jnp.where(degrees == dmin, lane, N))

    srow = jnp.sum(jnp.where(iota_r == seed, sim, 0.0), axis=0, keepdims=True)
    eye = iota_r == iota_c
    xcol = jnp.sum(jnp.where(eye, jnp.broadcast_to(srow, (N, N)), 0.0),
                   axis=1, keepdims=True)

    xi = jnp.broadcast_to(xcol, (N, N))
    xj = jnp.broadcast_to(srow, (N, N))
    gt = (xi > xj).astype(jnp.int32)
    eqlt = ((xi == xj) & (iota_r < iota_c)).astype(jnp.int32)
    rank = jnp.sum(gt + eqlt, axis=0, keepdims=True)

    member = (srow >= 0) | (lane == seed)
    v0 = jnp.where((rank < KTOP) & member, 1.0, 0.0)

    def prune_body(i, v):
        row = sim_ref[pl.ds(i, 1), :]
        s = jnp.sum(row * v)
        keepf = jnp.where(s > 0.0, 1.0, 0.0)
        return jnp.where(lane == i, v * keepf, v)

    v = jax.lax.fori_loop(0, N, prune_body, v0)

    mask = v > 0.0
    init = jnp.where(mask, lane + 1, 0)
    cfirst = (lane % HD) == 0
    clast = (lane % HD) == (HD - 1)
    bigv = jnp.full((1, N), BIG, jnp.int32)

    def cc_body(_, lab):
        l = jnp.where(mask, lab, BIG)
        tmp_ref[...] = bigv
        tmp_ref[:, HD:] = l[:, : N - HD]
        up = tmp_ref[...]
        tmp_ref[...] = bigv
        tmp_ref[:, : N - HD] = l[:, HD:]
        down = tmp_ref[...]
        tmp_ref[...] = bigv
        tmp_ref[:, 1:] = l[:, : N - 1]
        left = jnp.where(cfirst, BIG, tmp_ref[...])
        tmp_ref[...] = bigv
        tmp_ref[:, : N - 1] = l[:, 1:]
        right = jnp.where(clast, BIG, tmp_ref[...])
        m = jnp.minimum(jnp.minimum(l, up),
                        jnp.minimum(left, jnp.minimum(right, down)))
        return jnp.where(mask, m, 0)

    labels = jax.lax.fori_loop(0, 2 * HD, cc_body, init)
    out_ref[...] = labels


def _patch_feats(img):
    B, C, H, W_ = img.shape
    Hd, Wd = H // PATCH, W_ // PATCH
    x = img.reshape(B, C, Hd, PATCH, Wd, PATCH)
    x = jnp.transpose(x, (0, 2, 4, 3, 5, 1))
    return x.reshape(B, Hd * Wd, PATCH * PATCH * C)[0]


def _lost_call(feats, W, interpret=False):
    return pl.pallas_call(
        _lost_body,
        out_shape=jax.ShapeDtypeStruct((1, N), jnp.int32),
        scratch_shapes=[
            pltpu.VMEM((N, N), jnp.float32),
            pltpu.VMEM((1, N), jnp.int32),
        ],
        interpret=interpret,
    )(feats, W)


def kernel(img, W):
    feats = _patch_feats(img)
    labels = _lost_call(feats, W)
    return labels.reshape(HD, HD)


# speedup vs baseline: 48.1756x; 2.0723x over previous
import jax
import jax.numpy as jnp
from jax.experimental import pallas as pl
from jax.experimental.pallas import tpu as pltpu

PATCH = 16
KTOP = 100
N = 1024
HD = 32
BIG = HD * HD + 2


def _dense_body(feats_ref, w_ref, sim_ref, v0_ref):
    f32 = jnp.float32
    feats = feats_ref[...]
    w = w_ref[...]
    out = jax.lax.dot_general(
        feats, w, (((1,), (0,)), ((), ())),
        precision=jax.lax.Precision.HIGHEST, preferred_element_type=f32)
    sim = jax.lax.dot_general(
        out, out, (((1,), (1,)), ((), ())),
        precision=jax.lax.Precision.HIGHEST, preferred_element_type=f32)
    sim_ref[...] = sim

    iota_r = jax.lax.broadcasted_iota(jnp.int32, (N, N), 0)
    iota_c = jax.lax.broadcasted_iota(jnp.int32, (N, N), 1)
    lane = jax.lax.broadcasted_iota(jnp.int32, (1, N), 1)

    degrees = jnp.sum((sim >= 0).astype(jnp.int32), axis=0, keepdims=True)
    dmin = jnp.min(degrees)
    seed = jnp.min(jnp.where(degrees == dmin, lane, N))

    srow = jnp.sum(jnp.where(iota_r == seed, sim, 0.0), axis=0, keepdims=True)
    eye = iota_r == iota_c
    xcol = jnp.sum(jnp.where(eye, jnp.broadcast_to(srow, (N, N)), 0.0),
                   axis=1, keepdims=True)

    xi = jnp.broadcast_to(xcol, (N, N))
    xj = jnp.broadcast_to(srow, (N, N))
    gt = (xi > xj).astype(jnp.int32)
    eqlt = ((xi == xj) & (iota_r < iota_c)).astype(jnp.int32)
    rank = jnp.sum(gt + eqlt, axis=0, keepdims=True)

    member = (srow >= 0) | (lane == seed)
    v0_ref[...] = jnp.where((rank < KTOP) & member, 1, 0)


def _seq_body(v0s_ref, sim_ref, v0_ref, out_ref, v_ref, tmp_ref):
    lane = jax.lax.broadcasted_iota(jnp.int32, (1, N), 1)
    v_ref[...] = (v0_ref[...] != 0).astype(jnp.float32)

    def step(i, carry):
        @pl.when(v0s_ref[i] != 0)
        def _():
            row = sim_ref[pl.ds(i, 1), :]
            v = v_ref[...]
            s = jnp.sum(row * v)
            keepf = jnp.where(s > 0.0, 1.0, 0.0)
            v_ref[...] = jnp.where(lane == i, v * keepf, v)
        return carry

    jax.lax.fori_loop(0, N, step, 0)

    mask = v_ref[...] > 0.0
    init = jnp.where(mask, lane + 1, 0)
    cfirst = (lane % HD) == 0
    clast = (lane % HD) == (HD - 1)
    bigv = jnp.full((1, N), BIG, jnp.int32)

    def cc_body(_, lab):
        l = jnp.where(mask, lab, BIG)
        tmp_ref[...] = bigv
        tmp_ref[:, HD:] = l[:, : N - HD]
        up = tmp_ref[...]
        tmp_ref[...] = bigv
        tmp_ref[:, : N - HD] = l[:, HD:]
        down = tmp_ref[...]
        tmp_ref[...] = bigv
        tmp_ref[:, 1:] = l[:, : N - 1]
        left = jnp.where(cfirst, BIG, tmp_ref[...])
        tmp_ref[...] = bigv
        tmp_ref[:, : N - 1] = l[:, 1:]
        right = jnp.where(clast, BIG, tmp_ref[...])
        m = jnp.minimum(jnp.minimum(l, up),
                        jnp.minimum(left, jnp.minimum(right, down)))
        return jnp.where(mask, m, 0)

    out_ref[...] = jax.lax.fori_loop(0, 2 * HD, cc_body, init)


def _patch_feats(img):
    B, C, H, W_ = img.shape
    Hd, Wd = H // PATCH, W_ // PATCH
    x = img.reshape(B, C, Hd, PATCH, Wd, PATCH)
    x = jnp.transpose(x, (0, 2, 4, 3, 5, 1))
    return x.reshape(B, Hd * Wd, PATCH * PATCH * C)[0]


def _lost_call(feats, W, interpret=False):
    sim, v0 = pl.pallas_call(
        _dense_body,
        out_shape=(
            jax.ShapeDtypeStruct((N, N), jnp.float32),
            jax.ShapeDtypeStruct((1, N), jnp.int32),
        ),
        interpret=interpret,
    )(feats, W)

    labels = pl.pallas_call(
        _seq_body,
        out_shape=jax.ShapeDtypeStruct((1, N), jnp.int32),
        in_specs=[
            pl.BlockSpec(memory_space=pltpu.SMEM),
            pl.BlockSpec(memory_space=pltpu.VMEM),
            pl.BlockSpec(memory_space=pltpu.VMEM),
        ],
        scratch_shapes=[
            pltpu.VMEM((1, N), jnp.float32),
            pltpu.VMEM((1, N), jnp.int32),
        ],
        interpret=interpret,
    )(v0.reshape(N), sim, v0)
    return labels


def kernel(img, W):
    feats = _patch_feats(img)
    labels = _lost_call(feats, W)
    return labels.reshape(HD, HD)


# speedup vs baseline: 54.9254x; 1.1401x over previous
import jax
import jax.numpy as jnp
from jax.experimental import pallas as pl
from jax.experimental.pallas import tpu as pltpu

PATCH = 16
KTOP = 100
KPAD = 128
N = 1024
HD = 32
BIG = HD * HD + 2


def _dense_body(feats_ref, w_ref, sim_ref, v0_ref, act_ref):
    f32 = jnp.float32
    feats = feats_ref[...]
    w = w_ref[...]
    out = jax.lax.dot_general(
        feats, w, (((1,), (0,)), ((), ())),
        precision=jax.lax.Precision.HIGHEST, preferred_element_type=f32)
    sim = jax.lax.dot_general(
        out, out, (((1,), (1,)), ((), ())),
        precision=jax.lax.Precision.HIGHEST, preferred_element_type=f32)
    sim_ref[...] = sim

    iota_r = jax.lax.broadcasted_iota(jnp.int32, (N, N), 0)
    iota_c = jax.lax.broadcasted_iota(jnp.int32, (N, N), 1)
    lane = jax.lax.broadcasted_iota(jnp.int32, (1, N), 1)

    degrees = jnp.sum((sim >= 0).astype(jnp.int32), axis=0, keepdims=True)
    dmin = jnp.min(degrees)
    seed = jnp.min(jnp.where(degrees == dmin, lane, N))

    srow = jnp.sum(jnp.where(iota_r == seed, sim, 0.0), axis=0, keepdims=True)
    eye = iota_r == iota_c
    xcol = jnp.sum(jnp.where(eye, jnp.broadcast_to(srow, (N, N)), 0.0),
                   axis=1, keepdims=True)

    xi = jnp.broadcast_to(xcol, (N, N))
    xj = jnp.broadcast_to(srow, (N, N))
    gt = (xi > xj).astype(jnp.int32)
    eqlt = ((xi == xj) & (iota_r < iota_c)).astype(jnp.int32)
    rank = jnp.sum(gt + eqlt, axis=0, keepdims=True)

    member = (srow >= 0) | (lane == seed)
    v0row = jnp.where((rank < KTOP) & member, 1, 0)
    v0_ref[...] = v0row

    v0col = jnp.sum(jnp.where(eye, jnp.broadcast_to(v0row, (N, N)), 0),
                    axis=1, keepdims=True)
    v0col_b = jnp.broadcast_to(v0col, (N, N)) != 0
    pos_row = jnp.sum(((iota_r < iota_c) & v0col_b).astype(jnp.int32),
                      axis=0, keepdims=True)
    pos_col = jnp.sum(jnp.where(eye, jnp.broadcast_to(pos_row, (N, N)), 0),
                      axis=1, keepdims=True)
    kiota = jax.lax.broadcasted_iota(jnp.int32, (N, KPAD), 1)
    riota = jax.lax.broadcasted_iota(jnp.int32, (N, KPAD), 0)
    match = (jnp.broadcast_to(pos_col, (N, KPAD)) == kiota) & \
            (jnp.broadcast_to(v0col, (N, KPAD)) != 0)
    a_row = jnp.sum(jnp.where(match, riota, 0), axis=0, keepdims=True)
    lane_k = jax.lax.broadcasted_iota(jnp.int32, (1, KPAD), 1)
    cnt = jnp.sum(v0row)
    act_ref[...] = jnp.where(lane_k == KPAD - 1, cnt, a_row)


def _seq_body(act_ref, sim_ref, v0_ref, out_ref, v_ref, tmp_ref):
    lane = jax.lax.broadcasted_iota(jnp.int32, (1, N), 1)
    v_ref[...] = (v0_ref[...] != 0).astype(jnp.float32)
    cnt = act_ref[KPAD - 1]

    def step(k, carry):
        i = act_ref[k]
        row = sim_ref[pl.ds(i, 1), :]
        v = v_ref[...]
        s = jnp.sum(row * v, keepdims=True)
        keepf = jnp.where(s > 0.0, 1.0, 0.0)
        v_ref[...] = jnp.where(lane == i, v * keepf, v)
        return carry

    jax.lax.fori_loop(0, cnt, step, 0)

    mask = v_ref[...] > 0.0
    init = jnp.where(mask, lane + 1, 0)
    cfirst = (lane % HD) == 0
    clast = (lane % HD) == (HD - 1)
    bigv = jnp.full((1, N), BIG, jnp.int32)

    def cc_body(_, lab):
        l = jnp.where(mask, lab, BIG)
        tmp_ref[...] = bigv
        tmp_ref[:, HD:] = l[:, : N - HD]
        up = tmp_ref[...]
        tmp_ref[...] = bigv
        tmp_ref[:, : N - HD] = l[:, HD:]
        down = tmp_ref[...]
        tmp_ref[...] = bigv
        tmp_ref[:, 1:] = l[:, : N - 1]
        left = jnp.where(cfirst, BIG, tmp_ref[...])
        tmp_ref[...] = bigv
        tmp_ref[:, : N - 1] = l[:, 1:]
        right = jnp.where(clast, BIG, tmp_ref[...])
        m = jnp.minimum(jnp.minimum(l, up),
                        jnp.minimum(left, jnp.minimum(right, down)))
        return jnp.where(mask, m, 0)

    out_ref[...] = jax.lax.fori_loop(0, 2 * HD, cc_body, init)


def _patch_feats(img):
    B, C, H, W_ = img.shape
    Hd, Wd = H // PATCH, W_ // PATCH
    x = img.reshape(B, C, Hd, PATCH, Wd, PATCH)
    x = jnp.transpose(x, (0, 2, 4, 3, 5, 1))
    return x.reshape(B, Hd * Wd, PATCH * PATCH * C)[0]


def _lost_call(feats, W, interpret=False):
    sim, v0, act = pl.pallas_call(
        _dense_body,
        out_shape=(
            jax.ShapeDtypeStruct((N, N), jnp.float32),
            jax.ShapeDtypeStruct((1, N), jnp.int32),
            jax.ShapeDtypeStruct((1, KPAD), jnp.int32),
        ),
        interpret=interpret,
    )(feats, W)

    labels = pl.pallas_call(
        _seq_body,
        out_shape=jax.ShapeDtypeStruct((1, N), jnp.int32),
        in_specs=[
            pl.BlockSpec(memory_space=pltpu.SMEM),
            pl.BlockSpec(memory_space=pltpu.VMEM),
            pl.BlockSpec(memory_space=pltpu.VMEM),
        ],
        scratch_shapes=[
            pltpu.VMEM((1, N), jnp.float32),
            pltpu.VMEM((1, N), jnp.int32),
        ],
        interpret=interpret,
    )(act.reshape(KPAD), sim, v0)
    return labels


def kernel(img, W):
    feats = _patch_feats(img)
    labels = _lost_call(feats, W)
    return labels.reshape(HD, HD)


# speedup vs baseline: 61.4155x; 1.1182x over previous
import jax
import jax.numpy as jnp
from jax.experimental import pallas as pl
from jax.experimental.pallas import tpu as pltpu

PATCH = 16
KTOP = 100
KPAD = 128
N = 1024
HD = 32
BIG = HD * HD + 2


def _dense_body(feats_ref, w_ref, sim_ref, v0_ref, act_ref):
    f32 = jnp.float32
    feats = feats_ref[...]
    w = w_ref[...]
    out = jax.lax.dot_general(
        feats, w, (((1,), (0,)), ((), ())),
        precision=jax.lax.Precision.DEFAULT, preferred_element_type=f32)
    sim = jax.lax.dot_general(
        out, out, (((1,), (1,)), ((), ())),
        precision=jax.lax.Precision.DEFAULT, preferred_element_type=f32)
    sim_ref[...] = sim

    iota_r = jax.lax.broadcasted_iota(jnp.int32, (N, N), 0)
    iota_c = jax.lax.broadcasted_iota(jnp.int32, (N, N), 1)
    lane = jax.lax.broadcasted_iota(jnp.int32, (1, N), 1)

    degrees = jnp.sum((sim >= 0).astype(jnp.int32), axis=0, keepdims=True)
    dmin = jnp.min(degrees)
    seed = jnp.min(jnp.where(degrees == dmin, lane, N))

    srow = jnp.sum(jnp.where(iota_r == seed, sim, 0.0), axis=0, keepdims=True)
    eye = iota_r == iota_c
    xcol = jnp.sum(jnp.where(eye, jnp.broadcast_to(srow, (N, N)), 0.0),
                   axis=1, keepdims=True)

    xi = jnp.broadcast_to(xcol, (N, N))
    xj = jnp.broadcast_to(srow, (N, N))
    gt = (xi > xj).astype(jnp.int32)
    eqlt = ((xi == xj) & (iota_r < iota_c)).astype(jnp.int32)
    rank = jnp.sum(gt + eqlt, axis=0, keepdims=True)

    member = (srow >= 0) | (lane == seed)
    v0row = jnp.where((rank < KTOP) & member, 1, 0)
    v0_ref[...] = v0row

    v0col = jnp.sum(jnp.where(eye, jnp.broadcast_to(v0row, (N, N)), 0),
                    axis=1, keepdims=True)
    v0col_b = jnp.broadcast_to(v0col, (N, N)) != 0
    pos_row = jnp.sum(((iota_r < iota_c) & v0col_b).astype(jnp.int32),
                      axis=0, keepdims=True)
    pos_col = jnp.sum(jnp.where(eye, jnp.broadcast_to(pos_row, (N, N)), 0),
                      axis=1, keepdims=True)
    kiota = jax.lax.broadcasted_iota(jnp.int32, (N, KPAD), 1)
    riota = jax.lax.broadcasted_iota(jnp.int32, (N, KPAD), 0)
    match = (jnp.broadcast_to(pos_col, (N, KPAD)) == kiota) & \
            (jnp.broadcast_to(v0col, (N, KPAD)) != 0)
    a_row = jnp.sum(jnp.where(match, riota, 0), axis=0, keepdims=True)
    lane_k = jax.lax.broadcasted_iota(jnp.int32, (1, KPAD), 1)
    cnt = jnp.sum(v0row)
    act_ref[...] = jnp.where(lane_k == KPAD - 1, cnt, a_row)


def _seq_body(act_ref, sim_ref, v0_ref, out_ref, v_ref, tmp_ref):
    lane = jax.lax.broadcasted_iota(jnp.int32, (1, N), 1)
    v_ref[...] = (v0_ref[...] != 0).astype(jnp.float32)
    cnt = act_ref[KPAD - 1]

    def step(k, carry):
        i = act_ref[k]
        row = sim_ref[pl.ds(i, 1), :]
        v = v_ref[...]
        s = jnp.sum(row * v, keepdims=True)
        keepf = jnp.where(s > 0.0, 1.0, 0.0)
        v_ref[...] = jnp.where(lane == i, v * keepf, v)
        return carry

    jax.lax.fori_loop(0, cnt, step, 0)

    mask = v_ref[...] > 0.0
    init = jnp.where(mask, lane + 1, 0)
    cfirst = (lane % HD) == 0
    clast = (lane % HD) == (HD - 1)
    bigv = jnp.full((1, N), BIG, jnp.int32)

    def cc_body(_, lab):
        l = jnp.where(mask, lab, BIG)
        tmp_ref[...] = bigv
        tmp_ref[:, HD:] = l[:, : N - HD]
        up = tmp_ref[...]
        tmp_ref[...] = bigv
        tmp_ref[:, : N - HD] = l[:, HD:]
        down = tmp_ref[...]
        tmp_ref[...] = bigv
        tmp_ref[:, 1:] = l[:, : N - 1]
        left = jnp.where(cfirst, BIG, tmp_ref[...])
        tmp_ref[...] = bigv
        tmp_ref[:, : N - 1] = l[:, 1:]
        right = jnp.where(clast, BIG, tmp_ref[...])
        m = jnp.minimum(jnp.minimum(l, up),
                        jnp.minimum(left, jnp.minimum(right, down)))
        return jnp.where(mask, m, 0)

    out_ref[...] = jax.lax.fori_loop(0, 2 * HD, cc_body, init)


def _patch_feats(img):
    B, C, H, W_ = img.shape
    Hd, Wd = H // PATCH, W_ // PATCH
    x = img.reshape(B, C, Hd, PATCH, Wd, PATCH)
    x = jnp.transpose(x, (0, 2, 4, 3, 5, 1))
    return x.reshape(B, Hd * Wd, PATCH * PATCH * C)[0]


def _lost_call(feats, W, interpret=False):
    sim, v0, act = pl.pallas_call(
        _dense_body,
        out_shape=(
            jax.ShapeDtypeStruct((N, N), jnp.float32),
            jax.ShapeDtypeStruct((1, N), jnp.int32),
            jax.ShapeDtypeStruct((1, KPAD), jnp.int32),
        ),
        interpret=interpret,
    )(feats, W)

    labels = pl.pallas_call(
        _seq_body,
        out_shape=jax.ShapeDtypeStruct((1, N), jnp.int32),
        in_specs=[
            pl.BlockSpec(memory_space=pltpu.SMEM),
            pl.BlockSpec(memory_space=pltpu.VMEM),
            pl.BlockSpec(memory_space=pltpu.VMEM),
        ],
        scratch_shapes=[
            pltpu.VMEM((1, N), jnp.float32),
            pltpu.VMEM((1, N), jnp.int32),
        ],
        interpret=interpret,
    )(act.reshape(KPAD), sim, v0)
    return labels


def kernel(img, W):
    feats = _patch_feats(img)
    labels = _lost_call(feats, W)
    return labels.reshape(HD, HD)
